# baseline (speedup 1.0000x reference)
import functools

import jax
import jax.numpy as jnp
from jax import lax
from jax.experimental import pallas as pl
from jax.experimental.pallas import tpu as pltpu
from jax.experimental.pallas import tpu_sc as plsc

_K = 8192
_KBITS = 13
_DC = 32
_DW = 48
_D = 1024
_BL = 4 * 8192

_NC = 2
_NS = 16
_NW = _NC * _NS
_TPW = _BL // _NW
_CHUNK = 128
_NCHUNK = _TPW // _CHUNK

_TBLK = 2048
_NBLK = _BL // _TBLK


def _sc_gather_body(table_hbm, idx_hbm, y_hbm,
                    idx_v, d0_v, d1_v, rows0_v, rows1_v, gsem, ssem):
    c = lax.axis_index("c")
    s = lax.axis_index("s")
    wid = s * _NC + c
    pltpu.sync_copy(idx_hbm.at[pl.ds(wid * _NCHUNK, _NCHUNK)], idx_v)
    for j in range(_NCHUNK):
        for t in range(_CHUNK // 16):
            v = idx_v[j, pl.ds(t * 16, 16)]
            m = v < 4
            d0_v[j, pl.ds(t * 16, 16)] = jnp.where(m, 2 * _K + v, v & (_K - 1))
            d1_v[j, pl.ds(t * 16, 16)] = jnp.where(
                m, 2 * _K + 4, (v >> _KBITS) + _K)
    copies = []
    for j in range(_NCHUNK):
        copies.append(pltpu.async_copy(
            table_hbm.at[d0_v.at[j]],
            rows0_v.at[pl.ds(j * _CHUNK, _CHUNK)], gsem))
        copies.append(pltpu.async_copy(
            table_hbm.at[d1_v.at[j]],
            rows1_v.at[pl.ds(j * _CHUNK, _CHUNK)], gsem))
    base = wid * _TPW

    def _add_row(r, carry):
        for t in range(_DW // 16):
            rows0_v[r, pl.ds(t * 16, 16)] = (
                rows0_v[r, pl.ds(t * 16, 16)] + rows1_v[r, pl.ds(t * 16, 16)])
        return carry

    outs = []
    for j in range(_NCHUNK):
        copies[2 * j].wait()
        copies[2 * j + 1].wait()
        lax.fori_loop(j * _CHUNK, (j + 1) * _CHUNK, _add_row, 0)
        outs.append(pltpu.async_copy(
            rows0_v.at[pl.ds(j * _CHUNK, _CHUNK)],
            y_hbm.at[pl.ds(base + j * _CHUNK, _CHUNK)], ssem))
    for cp in outs:
        cp.wait()


@functools.cache
def _make_sc_gather():
    return functools.partial(
        pl.kernel,
        out_type=jax.ShapeDtypeStruct((_BL, _DW), jnp.float32),
        mesh=plsc.VectorSubcoreMesh(core_axis_name="c", subcore_axis_name="s",
                                    num_cores=_NC, num_subcores=_NS),
        scratch_types=[
            pltpu.VMEM((_NCHUNK, _CHUNK), jnp.int32),
            pltpu.VMEM((_NCHUNK, _CHUNK), jnp.int32),
            pltpu.VMEM((_NCHUNK, _CHUNK), jnp.int32),
            pltpu.VMEM((_TPW, _DW), jnp.float32),
            pltpu.VMEM((_TPW, _DW), jnp.float32),
            pltpu.SemaphoreType.DMA,
            pltpu.SemaphoreType.DMA,
        ],
        compiler_params=pltpu.CompilerParams(use_tc_tiling_on_sc=False),
    )(_sc_gather_body)


def _tc_body(y_ref, w_ref, out_ref):
    out_ref[...] = jnp.dot(y_ref[...].astype(jnp.bfloat16), w_ref[...],
                           preferred_element_type=jnp.float32)


def _build_table(codebooks):
    cb = codebooks.reshape(2 * _K, _DC)
    top = jnp.concatenate(
        [cb, jnp.zeros((2 * _K, _DW - _DC), cb.dtype)], axis=1)
    eye = jnp.eye(8, dtype=cb.dtype)[:, :8]
    eye = eye * (jnp.arange(8) < 4)[:, None].astype(cb.dtype)
    bottom = jnp.concatenate(
        [jnp.zeros((8, _DC), cb.dtype), eye,
         jnp.zeros((8, _DW - _DC - 8), cb.dtype)], axis=1)
    return jnp.concatenate([top, bottom], axis=0)


def kernel(idx, codebooks, Wdc, special_emb):
    idx = idx.astype(jnp.int32)
    table = _build_table(codebooks)
    idx_flat = idx.reshape(-1)
    y = _make_sc_gather()(table, idx_flat.reshape(_NW * _NCHUNK, _CHUNK))
    wcat = jnp.concatenate(
        [Wdc, special_emb,
         jnp.zeros((_DW - _DC - special_emb.shape[0], _D), Wdc.dtype)],
        axis=0).astype(jnp.bfloat16)
    out = pl.pallas_call(
        _tc_body,
        grid=(_NBLK,),
        in_specs=[
            pl.BlockSpec((_TBLK, _DW), lambda i: (i, 0)),
            pl.BlockSpec((_DW, _D), lambda i: (0, 0)),
        ],
        out_specs=pl.BlockSpec((_TBLK, _D), lambda i: (i, 0)),
        out_shape=jax.ShapeDtypeStruct((_BL, _D), jnp.float32),
    )(y, wcat)
    return out.reshape(idx.shape + (_D,))


# speedup vs baseline: 1.0908x; 1.0908x over previous
import functools

import jax
import jax.numpy as jnp
from jax import lax
from jax.experimental import pallas as pl
from jax.experimental.pallas import tpu as pltpu
from jax.experimental.pallas import tpu_sc as plsc

_K = 8192
_KBITS = 13
_DC = 32
_DW = 48
_D = 1024
_BL = 4 * 8192

_NC = 2
_NS = 16
_NW = _NC * _NS
_TPW = _BL // _NW
_CHUNK = 128
_NCHUNK = _TPW // _CHUNK

_TBLK = 2048
_NBLK = _BL // _TBLK


def _sc_gather_body(table_hbm, idx_hbm, y_hbm,
                    idx_v, d0_v, d1_v, rows0_v, rows1_v, gsem, ssem):
    c = lax.axis_index("c")
    s = lax.axis_index("s")
    wid = s * _NC + c
    row = wid // 8
    col = (wid % 8) * _TPW
    pltpu.sync_copy(idx_hbm.at[row, pl.ds(col, _TPW)], idx_v)
    for j in range(_NCHUNK):
        for t in range(_CHUNK // 16):
            v = idx_v[pl.ds(j * _CHUNK + t * 16, 16)]
            m = v < 4
            d0_v[j, pl.ds(t * 16, 16)] = jnp.where(m, 2 * _K + v, v & (_K - 1))
            d1_v[j, pl.ds(t * 16, 16)] = jnp.where(
                m, 2 * _K + 4, (v >> _KBITS) + _K)
    copies = []
    for j in range(_NCHUNK):
        copies.append(pltpu.async_copy(
            table_hbm.at[d0_v.at[j]],
            rows0_v.at[pl.ds(j * _CHUNK, _CHUNK)], gsem))
        copies.append(pltpu.async_copy(
            table_hbm.at[d1_v.at[j]],
            rows1_v.at[pl.ds(j * _CHUNK, _CHUNK)], gsem))
    base = wid * _TPW

    def _add_row(r, carry):
        for t in range(_DW // 16):
            rows0_v[r, pl.ds(t * 16, 16)] = (
                rows0_v[r, pl.ds(t * 16, 16)] + rows1_v[r, pl.ds(t * 16, 16)])
        return carry

    outs = []
    for j in range(_NCHUNK):
        copies[2 * j].wait()
        copies[2 * j + 1].wait()
        lax.fori_loop(j * _CHUNK, (j + 1) * _CHUNK, _add_row, 0)
        outs.append(pltpu.async_copy(
            rows0_v.at[pl.ds(j * _CHUNK, _CHUNK)],
            y_hbm.at[pl.ds(base + j * _CHUNK, _CHUNK)], ssem))
    for cp in outs:
        cp.wait()


@functools.cache
def _make_sc_gather():
    return functools.partial(
        pl.kernel,
        out_type=jax.ShapeDtypeStruct((_BL, _DW), jnp.float32),
        mesh=plsc.VectorSubcoreMesh(core_axis_name="c", subcore_axis_name="s",
                                    num_cores=_NC, num_subcores=_NS),
        scratch_types=[
            pltpu.VMEM((_TPW,), jnp.int32),
            pltpu.VMEM((_NCHUNK, _CHUNK), jnp.int32),
            pltpu.VMEM((_NCHUNK, _CHUNK), jnp.int32),
            pltpu.VMEM((_TPW, _DW), jnp.float32),
            pltpu.VMEM((_TPW, _DW), jnp.float32),
            pltpu.SemaphoreType.DMA,
            pltpu.SemaphoreType.DMA,
        ],
        compiler_params=pltpu.CompilerParams(use_tc_tiling_on_sc=False),
    )(_sc_gather_body)


def _tc_body(y_ref, w_ref, out_ref):
    out_ref[...] = jnp.dot(y_ref[...].astype(jnp.bfloat16), w_ref[...],
                           preferred_element_type=jnp.float32)


import numpy as _np

_BOTTOM = _np.zeros((8, _DW), _np.float32)
for _j in range(4):
    _BOTTOM[_j, _DC + _j] = 1.0


def _build_table(codebooks):
    cb = codebooks.reshape(2 * _K, _DC)
    top = jnp.pad(cb, ((0, 0), (0, _DW - _DC)))
    return jnp.concatenate([top, jnp.asarray(_BOTTOM)], axis=0)


def kernel(idx, codebooks, Wdc, special_emb):
    idx = idx.astype(jnp.int32)
    table = _build_table(codebooks)
    y = _make_sc_gather()(table, idx)
    wcat = jnp.concatenate(
        [Wdc, special_emb,
         jnp.zeros((_DW - _DC - special_emb.shape[0], _D), Wdc.dtype)],
        axis=0).astype(jnp.bfloat16)
    out = pl.pallas_call(
        _tc_body,
        grid=(_NBLK,),
        in_specs=[
            pl.BlockSpec((_TBLK, _DW), lambda i: (i, 0)),
            pl.BlockSpec((_DW, _D), lambda i: (0, 0)),
        ],
        out_specs=pl.BlockSpec((_TBLK, _D), lambda i: (i, 0)),
        out_shape=jax.ShapeDtypeStruct((_BL, _D), jnp.float32),
    )(y, wcat)
    return out.reshape(idx.shape + (_D,))


# speedup vs baseline: 1.0943x; 1.0032x over previous
import functools

import jax
import jax.numpy as jnp
from jax import lax
from jax.experimental import pallas as pl
from jax.experimental.pallas import tpu as pltpu
from jax.experimental.pallas import tpu_sc as plsc

_K = 8192
_KBITS = 13
_DC = 32
_DW = 48
_D = 1024
_BL = 4 * 8192

_NC = 2
_NS = 16
_NW = _NC * _NS
_TPW = _BL // _NW
_CHUNK = 128
_NCHUNK = _TPW // _CHUNK

_TBLK = 4096
_NBLK = _BL // _TBLK


def _sc_gather_body(table_hbm, idx_hbm, y_hbm,
                    idx_v, d0_v, d1_v, rows0_v, rows1_v, gsem, ssem):
    c = lax.axis_index("c")
    s = lax.axis_index("s")
    wid = s * _NC + c
    row = wid // 8
    col = (wid % 8) * _TPW
    pltpu.sync_copy(idx_hbm.at[row, pl.ds(col, _TPW)], idx_v)
    for j in range(_NCHUNK):
        for t in range(_CHUNK // 16):
            v = idx_v[pl.ds(j * _CHUNK + t * 16, 16)]
            m = v < 4
            d0_v[j, pl.ds(t * 16, 16)] = jnp.where(m, 2 * _K + v, v & (_K - 1))
            d1_v[j, pl.ds(t * 16, 16)] = jnp.where(
                m, 2 * _K + 4, (v >> _KBITS) + _K)
    copies = []
    for j in range(_NCHUNK):
        copies.append(pltpu.async_copy(
            table_hbm.at[d0_v.at[j]],
            rows0_v.at[pl.ds(j * _CHUNK, _CHUNK)], gsem))
        copies.append(pltpu.async_copy(
            table_hbm.at[d1_v.at[j]],
            rows1_v.at[pl.ds(j * _CHUNK, _CHUNK)], gsem))
    base = wid * _TPW

    def _add_row(r, carry):
        for t in range(_DW // 16):
            rows0_v[r, pl.ds(t * 16, 16)] = (
                rows0_v[r, pl.ds(t * 16, 16)] + rows1_v[r, pl.ds(t * 16, 16)])
        return carry

    outs = []
    for j in range(_NCHUNK):
        copies[2 * j].wait()
        copies[2 * j + 1].wait()
        lax.fori_loop(j * _CHUNK, (j + 1) * _CHUNK, _add_row, 0)
        outs.append(pltpu.async_copy(
            rows0_v.at[pl.ds(j * _CHUNK, _CHUNK)],
            y_hbm.at[pl.ds(base + j * _CHUNK, _CHUNK)], ssem))
    for cp in outs:
        cp.wait()


@functools.cache
def _make_sc_gather():
    return functools.partial(
        pl.kernel,
        out_type=jax.ShapeDtypeStruct((_BL, _DW), jnp.float32),
        mesh=plsc.VectorSubcoreMesh(core_axis_name="c", subcore_axis_name="s",
                                    num_cores=_NC, num_subcores=_NS),
        scratch_types=[
            pltpu.VMEM((_TPW,), jnp.int32),
            pltpu.VMEM((_NCHUNK, _CHUNK), jnp.int32),
            pltpu.VMEM((_NCHUNK, _CHUNK), jnp.int32),
            pltpu.VMEM((_TPW, _DW), jnp.float32),
            pltpu.VMEM((_TPW, _DW), jnp.float32),
            pltpu.SemaphoreType.DMA,
            pltpu.SemaphoreType.DMA,
        ],
        compiler_params=pltpu.CompilerParams(use_tc_tiling_on_sc=False),
    )(_sc_gather_body)


def _tc_body(y_ref, w_ref, out_ref):
    out_ref[...] = jnp.dot(y_ref[...].astype(jnp.bfloat16), w_ref[...],
                           preferred_element_type=jnp.float32)


import numpy as _np

_BOTTOM = _np.zeros((8, _DW), _np.float32)
for _j in range(4):
    _BOTTOM[_j, _DC + _j] = 1.0


def _build_table(codebooks):
    cb = codebooks.reshape(2 * _K, _DC)
    top = jnp.pad(cb, ((0, 0), (0, _DW - _DC)))
    return jnp.concatenate([top, jnp.asarray(_BOTTOM)], axis=0)


def kernel(idx, codebooks, Wdc, special_emb):
    idx = idx.astype(jnp.int32)
    table = _build_table(codebooks)
    y = _make_sc_gather()(table, idx)
    wcat = jnp.concatenate(
        [Wdc, special_emb,
         jnp.zeros((_DW - _DC - special_emb.shape[0], _D), Wdc.dtype)],
        axis=0).astype(jnp.bfloat16)
    out = pl.pallas_call(
        _tc_body,
        grid=(_NBLK,),
        in_specs=[
            pl.BlockSpec((_TBLK, _DW), lambda i: (i, 0)),
            pl.BlockSpec((_DW, _D), lambda i: (0, 0)),
        ],
        out_specs=pl.BlockSpec((_TBLK, _D), lambda i: (i, 0)),
        out_shape=jax.ShapeDtypeStruct((_BL, _D), jnp.float32),
    )(y, wcat)
    return out.reshape(idx.shape + (_D,))
